# baseline (speedup 1.0000x reference)
import functools

import jax
import jax.numpy as jnp
from jax import lax
from jax.experimental import pallas as pl
from jax.experimental.pallas import tpu as pltpu
from jax.experimental.pallas import tpu_sc as plsc

N = 10000
D = 128
E = 320000
G = 16

NC = 2
NS = 16
NW = NC * NS
K = 80
C = 125
EWP = C * K
E_PAD = NW * EWP
RPT = 632
N_PAD = NS * RPT

_f32 = jnp.float32


def _sc_agg_body(h_hbm, src_hbm, dst_hbm, zeros_hbm, out_hbm,
                 idx_s, idx_d, rb0, rb1, acc, sem0, sem1):
    c = lax.axis_index("c")
    s = lax.axis_index("s")
    w = c * NS + s

    pltpu.sync_copy(src_hbm.at[w], idx_s)
    pltpu.sync_copy(dst_hbm.at[w], idx_d)
    @pl.when(c == 0)
    def _():
        @pl.when(s < NS - 1)
        def _():
            pltpu.sync_copy(h_hbm.at[pl.ds(s * RPT, RPT)],
                            acc.at[pl.ds(s * RPT, RPT)])

        @pl.when(s == NS - 1)
        def _():
            last = N - (NS - 1) * RPT
            pltpu.sync_copy(h_hbm.at[pl.ds((NS - 1) * RPT, last)],
                            acc.at[pl.ds((NS - 1) * RPT, last)])
            pltpu.sync_copy(zeros_hbm.at[pl.ds(0, N_PAD - N)],
                            acc.at[pl.ds(N, N_PAD - N)])

    @pl.when(c == 1)
    def _():
        pltpu.sync_copy(zeros_hbm, acc.at[pl.ds(s * RPT, RPT)])

    plsc.subcore_barrier()

    def gather(j, rb, sem):
        pltpu.async_copy(h_hbm.at[idx_s.at[pl.ds(j * K, K)]], rb, sem)

    def gwait(rb, sem):
        pltpu.make_async_copy(h_hbm.at[idx_s.at[pl.ds(0, K)]], rb, sem).wait()

    gather(0, rb0, sem0)

    def pair(p, carry):
        gather(2 * p + 1, rb1, sem1)
        gwait(rb0, sem0)
        pltpu.sync_copy(rb0, acc.at[idx_d.at[2 * p]], add=True)
        gather(2 * p + 2, rb0, sem0)
        gwait(rb1, sem1)
        pltpu.sync_copy(rb1, acc.at[idx_d.at[2 * p + 1]], add=True)
        return carry

    lax.fori_loop(0, (C - 1) // 2, pair, 0)
    gwait(rb0, sem0)
    pltpu.sync_copy(rb0, acc.at[idx_d.at[C - 1]], add=True)
    plsc.subcore_barrier()
    pltpu.sync_copy(acc.at[pl.ds(s * RPT, RPT)],
                    out_hbm.at[c, pl.ds(s * RPT, RPT)])


_sc_agg = functools.partial(
    pl.kernel,
    mesh=plsc.VectorSubcoreMesh(core_axis_name="c", subcore_axis_name="s"),
    out_type=jax.ShapeDtypeStruct((NC, N_PAD, D), _f32),
    scratch_types=[
        pltpu.VMEM((EWP,), jnp.int32),
        pltpu.VMEM((C, K), jnp.int32),
        pltpu.VMEM((K, D), _f32),
        pltpu.VMEM((K, D), _f32),
        pltpu.VMEM_SHARED((N_PAD, D), _f32),
        pltpu.SemaphoreType.DMA,
        pltpu.SemaphoreType.DMA,
    ],
)(_sc_agg_body)


_BLK = 1000
_NB = N // _BLK


_DN_NT = (((1,), (1,)), ((), ()))


def _proj_body(x_ref, w_ref, b_ref, o_ref):
    z = lax.dot_general(x_ref[...], w_ref[...], _DN_NT,
                        preferred_element_type=_f32,
                        precision=lax.Precision.HIGHEST) + b_ref[...]
    o_ref[...] = jnp.maximum(z, 0.0)


def _layer_body(p_ref, w_ref, b_ref, g_ref, be_ref, o_ref):
    t = p_ref[0] + p_ref[1]
    z = lax.dot_general(t, w_ref[...], _DN_NT,
                        preferred_element_type=_f32,
                        precision=lax.Precision.HIGHEST) + b_ref[...]
    z = jnp.maximum(z, 0.0)
    mu = jnp.mean(z, axis=-1, keepdims=True)
    var = jnp.mean((z - mu) ** 2, axis=-1, keepdims=True)
    o_ref[...] = (z - mu) / jnp.sqrt(var + 1e-5) * g_ref[...] + be_ref[...]


def _final_body(p_ref, ids_ref, w_ref, b_ref, g_ref, be_ref,
                o_ref, sums, cnts):
    step = pl.program_id(0)

    @pl.when(step == 0)
    def _():
        sums[...] = jnp.zeros((G, D), _f32)
        cnts[...] = jnp.zeros((G, D), _f32)

    t = p_ref[0] + p_ref[1]
    z = lax.dot_general(t, w_ref[...], _DN_NT,
                        preferred_element_type=_f32,
                        precision=lax.Precision.HIGHEST) + b_ref[...]
    z = jnp.maximum(z, 0.0)
    mu = jnp.mean(z, axis=-1, keepdims=True)
    var = jnp.mean((z - mu) ** 2, axis=-1, keepdims=True)
    ln = (z - mu) / jnp.sqrt(var + 1e-5) * g_ref[...] + be_ref[...]

    ids = ids_ref[...]
    onehot = (lax.broadcasted_iota(jnp.int32, (_BLK, G), 1) == ids).astype(_f32)
    dn = (((0,), (0,)), ((), ()))
    sums[...] += lax.dot_general(onehot, ln, dn,
                                 preferred_element_type=_f32,
                                 precision=lax.Precision.HIGHEST)
    cnts[...] += lax.dot_general(onehot, jnp.ones((_BLK, D), _f32), dn,
                                 preferred_element_type=_f32,
                                 precision=lax.Precision.HIGHEST)
    o_ref[...] = sums[...] / jnp.maximum(cnts[...], 1.0)


_row_spec = pl.BlockSpec((_BLK, D), lambda i: (i, 0))
_p_spec = pl.BlockSpec((NC, _BLK, D), lambda i: (0, i, 0))
_w_spec = pl.BlockSpec((D, D), lambda i: (0, 0))
_v_spec = pl.BlockSpec((1, D), lambda i: (0, 0))

_proj = pl.pallas_call(
    _proj_body,
    grid=(_NB,),
    in_specs=[_row_spec, _w_spec, _v_spec],
    out_specs=_row_spec,
    out_shape=jax.ShapeDtypeStruct((N, D), _f32),
)

_layer = pl.pallas_call(
    _layer_body,
    grid=(_NB,),
    in_specs=[_p_spec, _w_spec, _v_spec, _v_spec, _v_spec],
    out_specs=_row_spec,
    out_shape=jax.ShapeDtypeStruct((N, D), _f32),
)

_final = pl.pallas_call(
    _final_body,
    grid=(_NB,),
    in_specs=[_p_spec, pl.BlockSpec((_BLK, 1), lambda i: (i, 0)),
              _w_spec, _v_spec, _v_spec, _v_spec],
    out_specs=pl.BlockSpec((G, D), lambda i: (0, 0)),
    out_shape=jax.ShapeDtypeStruct((G, D), _f32),
    scratch_shapes=[pltpu.VMEM((G, D), _f32), pltpu.VMEM((G, D), _f32)],
)


def kernel(x, edge_index, batch_ids, num_graphs, W_in, b_in,
           W1, b1, g1, be1, W2, b2, g2, be2, W3, b3, g3, be3):
    src3 = edge_index[0].reshape(NW, EWP)
    dst3 = edge_index[1].reshape(NW, C, K)
    zeros = jnp.zeros((RPT, D), _f32)
    ids2 = batch_ids.reshape(N, 1)

    h = _proj(x, W_in, b_in.reshape(1, D))
    for (W, b, gm, be) in ((W1, b1, g1, be1), (W2, b2, g2, be2)):
        p = _sc_agg(h, src3, dst3, zeros)
        h = _layer(p, W, b.reshape(1, D), gm.reshape(1, D),
                   be.reshape(1, D))
    p = _sc_agg(h, src3, dst3, zeros)
    return _final(p, ids2, W3, b3.reshape(1, D), g3.reshape(1, D),
                  be3.reshape(1, D))


# speedup vs baseline: 1.0392x; 1.0392x over previous
import functools

import jax
import jax.numpy as jnp
from jax import lax
from jax.experimental import pallas as pl
from jax.experimental.pallas import tpu as pltpu
from jax.experimental.pallas import tpu_sc as plsc

N = 10000
D = 128
E = 320000
G = 16

NC = 2
NS = 16
NW = NC * NS
K = 80
C = 125
EWP = C * K
E_PAD = NW * EWP
RPT = 632
N_PAD = NS * RPT

_f32 = jnp.float32


def _sc_agg_body(h_hbm, src_hbm, dst_hbm, zeros_hbm, out_hbm,
                 idx_s, idx_d, rb0, rb1, acc, sem0, sem1):
    c = lax.axis_index("c")
    s = lax.axis_index("s")
    w = c * NS + s

    pltpu.sync_copy(src_hbm.at[w], idx_s)
    pltpu.sync_copy(dst_hbm.at[w], idx_d)
    @pl.when(c == 0)
    def _():
        @pl.when(s < NS - 1)
        def _():
            pltpu.sync_copy(h_hbm.at[pl.ds(s * RPT, RPT)],
                            acc.at[pl.ds(s * RPT, RPT)])

        @pl.when(s == NS - 1)
        def _():
            last = N - (NS - 1) * RPT
            pltpu.sync_copy(h_hbm.at[pl.ds((NS - 1) * RPT, last)],
                            acc.at[pl.ds((NS - 1) * RPT, last)])
            pltpu.sync_copy(zeros_hbm.at[pl.ds(0, N_PAD - N)],
                            acc.at[pl.ds(N, N_PAD - N)])

    @pl.when(c == 1)
    def _():
        pltpu.sync_copy(zeros_hbm, acc.at[pl.ds(s * RPT, RPT)])

    plsc.subcore_barrier()

    def gather(j, rb, sem):
        pltpu.async_copy(h_hbm.at[idx_s.at[pl.ds(j * K, K)]], rb, sem)

    def gwait(rb, sem):
        pltpu.make_async_copy(h_hbm.at[idx_s.at[pl.ds(0, K)]], rb, sem).wait()

    gather(0, rb0, sem0)

    def pair(p, carry):
        gather(2 * p + 1, rb1, sem1)
        gwait(rb0, sem0)
        pltpu.sync_copy(rb0, acc.at[idx_d.at[2 * p]], add=True)
        gather(2 * p + 2, rb0, sem0)
        gwait(rb1, sem1)
        pltpu.sync_copy(rb1, acc.at[idx_d.at[2 * p + 1]], add=True)
        return carry

    lax.fori_loop(0, (C - 1) // 2, pair, 0)
    gwait(rb0, sem0)
    pltpu.sync_copy(rb0, acc.at[idx_d.at[C - 1]], add=True)
    plsc.subcore_barrier()
    pltpu.sync_copy(acc.at[pl.ds(s * RPT, RPT)],
                    out_hbm.at[c, pl.ds(s * RPT, RPT)])


_sc_agg = functools.partial(
    pl.kernel,
    mesh=plsc.VectorSubcoreMesh(core_axis_name="c", subcore_axis_name="s"),
    out_type=jax.ShapeDtypeStruct((NC, N_PAD, D), _f32),
    scratch_types=[
        pltpu.VMEM((EWP,), jnp.int32),
        pltpu.VMEM((C, K), jnp.int32),
        pltpu.VMEM((K, D), _f32),
        pltpu.VMEM((K, D), _f32),
        pltpu.VMEM_SHARED((N_PAD, D), _f32),
        pltpu.SemaphoreType.DMA,
        pltpu.SemaphoreType.DMA,
    ],
)(_sc_agg_body)


_BLK = 2000
_NB = N // _BLK


_DN_NT = (((1,), (1,)), ((), ()))


def _proj_body(x_ref, w_ref, b_ref, o_ref):
    z = lax.dot_general(x_ref[...], w_ref[...], _DN_NT,
                        preferred_element_type=_f32,
                        precision=lax.Precision.HIGHEST) + b_ref[...]
    o_ref[...] = jnp.maximum(z, 0.0)


def _layer_body(p_ref, w_ref, b_ref, g_ref, be_ref, o_ref):
    t = p_ref[0] + p_ref[1]
    z = lax.dot_general(t, w_ref[...], _DN_NT,
                        preferred_element_type=_f32,
                        precision=lax.Precision.HIGHEST) + b_ref[...]
    z = jnp.maximum(z, 0.0)
    mu = jnp.mean(z, axis=-1, keepdims=True)
    var = jnp.mean((z - mu) ** 2, axis=-1, keepdims=True)
    o_ref[...] = (z - mu) / jnp.sqrt(var + 1e-5) * g_ref[...] + be_ref[...]


def _final_body(p_ref, ids_ref, w_ref, b_ref, g_ref, be_ref,
                o_ref, sums, cnts):
    step = pl.program_id(0)

    @pl.when(step == 0)
    def _():
        sums[...] = jnp.zeros((G, D), _f32)
        cnts[...] = jnp.zeros((G, D), _f32)

    t = p_ref[0] + p_ref[1]
    z = lax.dot_general(t, w_ref[...], _DN_NT,
                        preferred_element_type=_f32,
                        precision=lax.Precision.HIGHEST) + b_ref[...]
    z = jnp.maximum(z, 0.0)
    mu = jnp.mean(z, axis=-1, keepdims=True)
    var = jnp.mean((z - mu) ** 2, axis=-1, keepdims=True)
    ln = (z - mu) / jnp.sqrt(var + 1e-5) * g_ref[...] + be_ref[...]

    ids = ids_ref[...]
    onehot = (lax.broadcasted_iota(jnp.int32, (_BLK, G), 1) == ids).astype(_f32)
    dn = (((0,), (0,)), ((), ()))
    sums[...] += lax.dot_general(onehot, ln, dn,
                                 preferred_element_type=_f32,
                                 precision=lax.Precision.HIGHEST)
    cnts[...] += lax.dot_general(onehot, jnp.ones((_BLK, D), _f32), dn,
                                 preferred_element_type=_f32,
                                 precision=lax.Precision.HIGHEST)
    o_ref[...] = sums[...] / jnp.maximum(cnts[...], 1.0)


_row_spec = pl.BlockSpec((_BLK, D), lambda i: (i, 0))
_p_spec = pl.BlockSpec((NC, _BLK, D), lambda i: (0, i, 0))
_w_spec = pl.BlockSpec((D, D), lambda i: (0, 0))
_v_spec = pl.BlockSpec((1, D), lambda i: (0, 0))

_proj = pl.pallas_call(
    _proj_body,
    grid=(_NB,),
    in_specs=[_row_spec, _w_spec, _v_spec],
    out_specs=_row_spec,
    out_shape=jax.ShapeDtypeStruct((N, D), _f32),
)

_layer = pl.pallas_call(
    _layer_body,
    grid=(_NB,),
    in_specs=[_p_spec, _w_spec, _v_spec, _v_spec, _v_spec],
    out_specs=_row_spec,
    out_shape=jax.ShapeDtypeStruct((N, D), _f32),
)

_final = pl.pallas_call(
    _final_body,
    grid=(_NB,),
    in_specs=[_p_spec, pl.BlockSpec((_BLK, 1), lambda i: (i, 0)),
              _w_spec, _v_spec, _v_spec, _v_spec],
    out_specs=pl.BlockSpec((G, D), lambda i: (0, 0)),
    out_shape=jax.ShapeDtypeStruct((G, D), _f32),
    scratch_shapes=[pltpu.VMEM((G, D), _f32), pltpu.VMEM((G, D), _f32)],
)


def kernel(x, edge_index, batch_ids, num_graphs, W_in, b_in,
           W1, b1, g1, be1, W2, b2, g2, be2, W3, b3, g3, be3):
    src3 = edge_index[0].reshape(NW, EWP)
    dst3 = edge_index[1].reshape(NW, C, K)
    zeros = jnp.zeros((RPT, D), _f32)
    ids2 = batch_ids.reshape(N, 1)

    h = _proj(x, W_in, b_in.reshape(1, D))
    for (W, b, gm, be) in ((W1, b1, g1, be1), (W2, b2, g2, be2)):
        p = _sc_agg(h, src3, dst3, zeros)
        h = _layer(p, W, b.reshape(1, D), gm.reshape(1, D),
                   be.reshape(1, D))
    p = _sc_agg(h, src3, dst3, zeros)
    return _final(p, ids2, W3, b3.reshape(1, D), g3.reshape(1, D),
                  be3.reshape(1, D))


# speedup vs baseline: 1.0419x; 1.0027x over previous
import functools

import jax
import jax.numpy as jnp
from jax import lax
from jax.experimental import pallas as pl
from jax.experimental.pallas import tpu as pltpu
from jax.experimental.pallas import tpu_sc as plsc

N = 10000
D = 128
E = 320000
G = 16

NC = 2
NS = 16
NW = NC * NS
K = 80
C = 125
EWP = C * K
RPT = 632
N_PAD = NS * RPT

_f32 = jnp.float32


def _sc_agg_body(h_hbm, src_hbm, dst_hbm, zeros_hbm, out_hbm,
                 idx_s, idx_d, rb0, rb1, acc, sem0, sem1):
    c = lax.axis_index("c")
    s = lax.axis_index("s")
    w = c * NS + s

    pltpu.sync_copy(src_hbm.at[w], idx_s)
    pltpu.sync_copy(dst_hbm.at[w], idx_d)
    @pl.when(c == 0)
    def _():
        @pl.when(s < NS - 1)
        def _():
            pltpu.sync_copy(h_hbm.at[pl.ds(s * RPT, RPT)],
                            acc.at[pl.ds(s * RPT, RPT)])

        @pl.when(s == NS - 1)
        def _():
            last = N - (NS - 1) * RPT
            pltpu.sync_copy(h_hbm.at[pl.ds((NS - 1) * RPT, last)],
                            acc.at[pl.ds((NS - 1) * RPT, last)])
            pltpu.sync_copy(zeros_hbm.at[pl.ds(0, N_PAD - N)],
                            acc.at[pl.ds(N, N_PAD - N)])

    @pl.when(c == 1)
    def _():
        pltpu.sync_copy(zeros_hbm, acc.at[pl.ds(s * RPT, RPT)])

    plsc.subcore_barrier()

    def gather(j, rb, sem):
        pltpu.async_copy(h_hbm.at[idx_s.at[pl.ds(j * K, K)]], rb, sem)

    def gwait(rb, sem):
        pltpu.make_async_copy(h_hbm.at[idx_s.at[pl.ds(0, K)]], rb, sem).wait()

    gather(0, rb0, sem0)

    def pair(p, carry):
        gather(2 * p + 1, rb1, sem1)
        gwait(rb0, sem0)
        pltpu.sync_copy(rb0, acc.at[idx_d.at[2 * p]], add=True)
        gather(2 * p + 2, rb0, sem0)
        gwait(rb1, sem1)
        pltpu.sync_copy(rb1, acc.at[idx_d.at[2 * p + 1]], add=True)
        return carry

    lax.fori_loop(0, (C - 1) // 2, pair, 0)
    gwait(rb0, sem0)
    pltpu.sync_copy(rb0, acc.at[idx_d.at[C - 1]], add=True)
    plsc.subcore_barrier()
    pltpu.sync_copy(acc.at[pl.ds(s * RPT, RPT)],
                    out_hbm.at[c, pl.ds(s * RPT, RPT)])


_sc_agg = functools.partial(
    pl.kernel,
    mesh=plsc.VectorSubcoreMesh(core_axis_name="c", subcore_axis_name="s"),
    out_type=jax.ShapeDtypeStruct((NC, N_PAD, D), _f32),
    scratch_types=[
        pltpu.VMEM((EWP,), jnp.int32),
        pltpu.VMEM((C, K), jnp.int32),
        pltpu.VMEM((K, D), _f32),
        pltpu.VMEM((K, D), _f32),
        pltpu.VMEM_SHARED((N_PAD, D), _f32),
        pltpu.SemaphoreType.DMA,
        pltpu.SemaphoreType.DMA,
    ],
)(_sc_agg_body)


_BLK = 2000
_NB = N // _BLK


_DN_NT = (((1,), (1,)), ((), ()))


def _proj_body(x_ref, w_ref, b_ref, o_ref):
    z = lax.dot_general(x_ref[...], w_ref[...], _DN_NT,
                        preferred_element_type=_f32,
                        precision=lax.Precision.HIGHEST) + b_ref[...]
    o_ref[...] = jnp.maximum(z, 0.0)


def _layer_body(p_ref, w_ref, b_ref, g_ref, be_ref, o_ref):
    t = p_ref[0] + p_ref[1]
    z = lax.dot_general(t, w_ref[...], _DN_NT,
                        preferred_element_type=_f32,
                        precision=lax.Precision.HIGHEST) + b_ref[...]
    z = jnp.maximum(z, 0.0)
    mu = jnp.mean(z, axis=-1, keepdims=True)
    var = jnp.mean((z - mu) ** 2, axis=-1, keepdims=True)
    o_ref[...] = (z - mu) / jnp.sqrt(var + 1e-5) * g_ref[...] + be_ref[...]


def _final_body(p_ref, ids_ref, w_ref, b_ref, g_ref, be_ref,
                o_ref, sums, cnts):
    step = pl.program_id(0)

    @pl.when(step == 0)
    def _():
        sums[...] = jnp.zeros((G, D), _f32)
        cnts[...] = jnp.zeros((G, D), _f32)

    t = p_ref[0] + p_ref[1]
    z = lax.dot_general(t, w_ref[...], _DN_NT,
                        preferred_element_type=_f32,
                        precision=lax.Precision.HIGHEST) + b_ref[...]
    z = jnp.maximum(z, 0.0)
    mu = jnp.mean(z, axis=-1, keepdims=True)
    var = jnp.mean((z - mu) ** 2, axis=-1, keepdims=True)
    ln = (z - mu) / jnp.sqrt(var + 1e-5) * g_ref[...] + be_ref[...]

    ids = ids_ref[...]
    onehot = (lax.broadcasted_iota(jnp.int32, (_BLK, G), 1) == ids).astype(_f32)
    dn = (((0,), (0,)), ((), ()))
    sums[...] += lax.dot_general(onehot, ln, dn,
                                 preferred_element_type=_f32,
                                 precision=lax.Precision.HIGHEST)
    cnts[...] += lax.dot_general(onehot, jnp.ones((_BLK, D), _f32), dn,
                                 preferred_element_type=_f32,
                                 precision=lax.Precision.HIGHEST)
    o_ref[...] = sums[...] / jnp.maximum(cnts[...], 1.0)


_row_spec = pl.BlockSpec((_BLK, D), lambda i: (i, 0))
_p_spec = pl.BlockSpec((NC, _BLK, D), lambda i: (0, i, 0))
_w_spec = pl.BlockSpec((D, D), lambda i: (0, 0))
_v_spec = pl.BlockSpec((1, D), lambda i: (0, 0))

_proj = pl.pallas_call(
    _proj_body,
    grid=(_NB,),
    in_specs=[_row_spec, _w_spec, _v_spec],
    out_specs=_row_spec,
    out_shape=jax.ShapeDtypeStruct((N, D), _f32),
)

_layer = pl.pallas_call(
    _layer_body,
    grid=(_NB,),
    in_specs=[_p_spec, _w_spec, _v_spec, _v_spec, _v_spec],
    out_specs=_row_spec,
    out_shape=jax.ShapeDtypeStruct((N, D), _f32),
)

_final = pl.pallas_call(
    _final_body,
    grid=(_NB,),
    in_specs=[_p_spec, pl.BlockSpec((_BLK, 1), lambda i: (i, 0)),
              _w_spec, _v_spec, _v_spec, _v_spec],
    out_specs=pl.BlockSpec((G, D), lambda i: (0, 0)),
    out_shape=jax.ShapeDtypeStruct((G, D), _f32),
    scratch_shapes=[pltpu.VMEM((G, D), _f32), pltpu.VMEM((G, D), _f32)],
)


def kernel(x, edge_index, batch_ids, num_graphs, W_in, b_in,
           W1, b1, g1, be1, W2, b2, g2, be2, W3, b3, g3, be3):
    src3 = edge_index[0].reshape(NW, EWP)
    dst3 = edge_index[1].reshape(NW, C, K)
    zeros = jnp.zeros((RPT, D), _f32)
    ids2 = batch_ids.reshape(N, 1)

    h = _proj(x, W_in, b_in.reshape(1, D))
    for (W, b, gm, be) in ((W1, b1, g1, be1), (W2, b2, g2, be2)):
        p = _sc_agg(h, src3, dst3, zeros)
        h = _layer(p, W, b.reshape(1, D), gm.reshape(1, D),
                   be.reshape(1, D))
    p = _sc_agg(h, src3, dst3, zeros)
    return _final(p, ids2, W3, b3.reshape(1, D), g3.reshape(1, D),
                  be3.reshape(1, D))
